# baseline (speedup 1.0000x reference)
import jax
import jax.numpy as jnp
from jax.experimental import pallas as pl
from jax.experimental.pallas import tpu as pltpu


def _make_se_kernel(c, h, w, n_s):
    inv_hw = 1.0 / float(h * w)
    rows_half = c * h // n_s
    c_half = c // n_s

    def _body(x_ref, w1t_ref, w2_ref, o_ref, s_ref):
        k = pl.program_id(1)

        @pl.when(k == 0)
        def _():
            xv = x_ref[...].reshape(c, h, w)
            pooled = jnp.sum(xv, axis=(1, 2), keepdims=True)[..., 0]
            pooled = pooled * inv_hw
            hid = jnp.sum(w1t_ref[...] * pooled, axis=0, keepdims=True)
            hid = jnp.maximum(hid, 0.0)
            s = jnp.sum(w2_ref[...] * hid, axis=-1, keepdims=True)
            s_ref[...] = jax.nn.sigmoid(s)

        xh = x_ref[pl.ds(k * rows_half, rows_half), :].reshape(c_half, h, w)
        s = s_ref[pl.ds(k * c_half, c_half), :]
        o_ref[...] = (xh * s[:, :, None]).reshape(rows_half, w).astype(o_ref.dtype)

    return _body


def kernel(x, w1, w2):
    b, c, h, w = x.shape
    hidden = w1.shape[0]

    n_s = 2 if c % 2 == 0 else 1

    x2 = x.astype(jnp.float32).reshape(b * c * h, w)
    w1t = jnp.transpose(w1.astype(jnp.float32))
    w2f = w2.astype(jnp.float32)

    out = pl.pallas_call(
        _make_se_kernel(c, h, w, n_s),
        out_shape=jax.ShapeDtypeStruct((b * c * h, w), x.dtype),
        grid=(b, n_s),
        in_specs=[
            pl.BlockSpec((c * h, w), lambda i, k: (i, 0)),
            pl.BlockSpec((c, hidden), lambda i, k: (0, 0)),
            pl.BlockSpec((c, hidden), lambda i, k: (0, 0)),
        ],
        out_specs=pl.BlockSpec((c * h // n_s, w), lambda i, k: (n_s * i + k, 0)),
        scratch_shapes=[pltpu.VMEM((c, 1), jnp.float32)],
        compiler_params=pltpu.CompilerParams(
            dimension_semantics=("arbitrary", "arbitrary"),
            vmem_limit_bytes=100 * 1024 * 1024,
        ),
    )(x2, w1t, w2f)

    return out.reshape(b, c, h, w)
